# baseline (speedup 1.0000x reference)
import jax
import jax.numpy as jnp
from jax import lax
from jax.experimental import pallas as pl
from jax.experimental.pallas import tpu as pltpu
from jax.experimental.pallas import tpu_sc as plsc

_D = 20
_L = 16
_NW = 32
_C = 1024
_G = 128


def _body(len_hbm, w_hbm, out_hbm, len_v, idx_v, rows_v, sem):
    n = len_hbm.shape[0]
    per_w = n // _NW
    wid = lax.axis_index("s") * 2 + lax.axis_index("c")
    base = wid * per_w

    def chunk(ci, _):
        row0 = base + ci * _C
        pltpu.sync_copy(len_hbm.at[pl.ds(row0, _C)], len_v)

        @plsc.parallel_loop(0, _C // _L, step=1, unroll=4)
        def bucketize(gi):
            r0 = gi * _L
            l = len_v[pl.ds(r0, _L)]
            idx_v[pl.ds(r0, _L)] = jnp.minimum(jnp.maximum(l + 4, 0), 7)

        copies = [
            pltpu.async_copy(
                w_hbm.at[idx_v.at[pl.ds(k * _G, _G)]],
                rows_v.at[pl.ds(k * _G, _G)],
                sem,
            )
            for k in range(_C // _G)
        ]
        for c in copies:
            c.wait()

        pltpu.sync_copy(rows_v, out_hbm.at[pl.ds(row0, _C)])
        return 0

    lax.fori_loop(0, per_w // _C, chunk, 0)


def kernel(lengths, W):
    n = lengths.shape[0]
    lengths = lengths.astype(jnp.int32)
    W = W.astype(jnp.float32)

    mesh = plsc.VectorSubcoreMesh(core_axis_name="c", subcore_axis_name="s")
    out = pl.kernel(
        _body,
        out_type=jax.ShapeDtypeStruct((n, _D), jnp.float32),
        mesh=mesh,
        compiler_params=pltpu.CompilerParams(
            needs_layout_passes=False, use_tc_tiling_on_sc=False
        ),
        scratch_types=[
            pltpu.VMEM((_C,), jnp.int32),
            pltpu.VMEM((_C,), jnp.int32),
            pltpu.VMEM((_C, _D), jnp.float32),
            pltpu.SemaphoreType.DMA,
        ],
    )(lengths, W)
    return out


# speedup vs baseline: 5.0439x; 5.0439x over previous
import jax
import jax.numpy as jnp
import numpy as np
from jax import lax
from jax.experimental import pallas as pl
from jax.experimental.pallas import tpu as pltpu
from jax.experimental.pallas import tpu_sc as plsc

_D = 20
_L = 16
_NW = 32
_C = 1024
_NP = 5

_ROW_PAT = [tuple((t * _L + q) // _D for q in range(_L)) for t in range(_NP)]
_COL_PAT = [tuple((t * _L + q) % _D for q in range(_L)) for t in range(_NP)]


def _body(len_hbm, w_hbm, out_hbm, tab_v, len_v, idx_v, out_v):
    n = len_hbm.shape[0]
    per_w = n // _NW
    wid = lax.axis_index("s") * 2 + lax.axis_index("c")
    base = wid * per_w

    pltpu.sync_copy(w_hbm, tab_v)

    row_pat, col_pat = [], []
    for t in range(_NP):
        p = t * _L + lax.iota(jnp.int32, _L)
        i = lax.div(p, jnp.int32(_D))
        row_pat.append(i)
        col_pat.append(p - i * _D)

    def chunk(ci, _):
        row0 = base + ci * _C
        pltpu.sync_copy(len_hbm.at[pl.ds(row0, _C)], len_v)

        @plsc.parallel_loop(0, _C // _L, step=1, unroll=4)
        def bucketize(gi):
            r0 = gi * _L
            l = len_v[pl.ds(r0, _L)]
            idx_v[pl.ds(r0, _L)] = jnp.minimum(jnp.maximum(l + 4, 0), 7)

        @plsc.parallel_loop(0, _C // 4, step=1, unroll=2)
        def emit(q):
            p0 = q * (_NP * _L)
            r0 = q * 4
            for t in range(_NP):
                ivec = r0 + row_pat[t]
                e = plsc.load_gather(idx_v, [ivec])
                v = plsc.load_gather(tab_v, [e, col_pat[t]])
                out_v[pl.ds(p0 + t * _L, _L)] = v

        pltpu.sync_copy(out_v, out_hbm.at[pl.ds(row0 * _D, _C * _D)])
        return 0

    lax.fori_loop(0, per_w // _C, chunk, 0)


def kernel(lengths, W):
    n = lengths.shape[0]
    lengths = lengths.astype(jnp.int32)
    W = W.astype(jnp.float32)

    mesh = plsc.VectorSubcoreMesh(core_axis_name="c", subcore_axis_name="s")
    out = pl.kernel(
        _body,
        out_type=jax.ShapeDtypeStruct((n * _D,), jnp.float32),
        mesh=mesh,
        compiler_params=pltpu.CompilerParams(needs_layout_passes=False),
        scratch_types=[
            pltpu.VMEM((8, _D), jnp.float32),
            pltpu.VMEM((_C,), jnp.int32),
            pltpu.VMEM((_C,), jnp.int32),
            pltpu.VMEM((_C * _D,), jnp.float32),
        ],
    )(lengths, W)
    return out.reshape(n, _D)


# speedup vs baseline: 65.7670x; 13.0388x over previous
import jax
import jax.numpy as jnp
from jax import lax
from jax.experimental import pallas as pl
from jax.experimental.pallas import tpu as pltpu
from jax.experimental.pallas import tpu_sc as plsc

_D = 20
_L = 16
_NW = 32
_C = 1024


def _body(len_hbm, wt_hbm, out_hbm, tab_v, len_v, out_v):
    n = len_hbm.shape[0]
    per_w = n // _NW
    wid = lax.axis_index("s") * 2 + lax.axis_index("c")
    base = wid * per_w

    pltpu.sync_copy(wt_hbm, tab_v)

    def chunk(ci, _):
        row0 = base + ci * _C
        pltpu.sync_copy(len_hbm.at[pl.ds(row0, _C)], len_v)

        @plsc.parallel_loop(0, _C // _L, step=1, unroll=2)
        def emit(gi):
            r0 = gi * _L
            l = len_v[pl.ds(r0, _L)]
            e = jnp.minimum(jnp.maximum(l + 4, 0), 7)
            for j in range(_D):
                v = plsc.load_gather(tab_v, [e + j * 8])
                out_v[j, pl.ds(r0, _L)] = v

        pltpu.sync_copy(out_v, out_hbm.at[:, pl.ds(row0, _C)])
        return 0

    lax.fori_loop(0, per_w // _C, chunk, 0)


def kernel(lengths, W):
    n = lengths.shape[0]
    lengths = lengths.astype(jnp.int32)
    wt = W.astype(jnp.float32).T.reshape(-1)

    mesh = plsc.VectorSubcoreMesh(core_axis_name="c", subcore_axis_name="s")
    out_t = pl.kernel(
        _body,
        out_type=jax.ShapeDtypeStruct((_D, n), jnp.float32),
        mesh=mesh,
        compiler_params=pltpu.CompilerParams(needs_layout_passes=False),
        scratch_types=[
            pltpu.VMEM((_D * 8,), jnp.float32),
            pltpu.VMEM((_C,), jnp.int32),
            pltpu.VMEM((_D, _C), jnp.float32),
        ],
    )(lengths, wt)
    return out_t.T


# speedup vs baseline: 129.2621x; 1.9655x over previous
import jax
import jax.numpy as jnp
from jax import lax
from jax.experimental import pallas as pl
from jax.experimental.pallas import tpu as pltpu
from jax.experimental.pallas import tpu_sc as plsc

_D = 20
_L = 16
_NW = 32
_C = 2048


def _body(len_hbm, wt_hbm, out_hbm, tab_v, len_v, out_v,
          si0, si1, so0, so1):
    n = len_hbm.shape[0]
    per_w = n // _NW
    nc = per_w // _C
    npairs = nc // 2
    wid = lax.axis_index("s") * 2 + lax.axis_index("c")
    base = wid * per_w
    sin = (si0, si1)
    sout = (so0, so1)

    pltpu.sync_copy(wt_hbm, tab_v)

    for b in (0, 1):
        pltpu.async_copy(
            len_hbm.at[pl.ds(base + b * _C, _C)], len_v.at[b], sin[b])

    def pair(ci2, _):
        for b in (0, 1):
            ci = ci2 * 2 + b
            row0 = base + ci * _C

            pltpu.make_async_copy(
                len_hbm.at[pl.ds(row0, _C)], len_v.at[b], sin[b]).wait()

            @pl.when(ci2 > 0)
            def _():
                pltpu.make_async_copy(
                    out_v.at[b],
                    out_hbm.at[:, pl.ds(row0 - 2 * _C, _C)],
                    sout[b]).wait()

            @plsc.parallel_loop(0, _C // _L, step=1, unroll=2)
            def emit(gi):
                r0 = gi * _L
                l = len_v[b, pl.ds(r0, _L)]
                e = jnp.minimum(jnp.maximum(l + 4, 0), 7)
                for j in range(_D):
                    v = plsc.load_gather(tab_v, [e + j * 8])
                    out_v[b, j, pl.ds(r0, _L)] = v

            pltpu.async_copy(
                out_v.at[b], out_hbm.at[:, pl.ds(row0, _C)], sout[b])

            @pl.when(ci2 < npairs - 1)
            def _():
                pltpu.async_copy(
                    len_hbm.at[pl.ds(row0 + 2 * _C, _C)],
                    len_v.at[b], sin[b])
        return 0

    lax.fori_loop(0, npairs, pair, 0)

    for b in (0, 1):
        pltpu.make_async_copy(
            out_v.at[b],
            out_hbm.at[:, pl.ds(base + (nc - 2 + b) * _C, _C)],
            sout[b]).wait()


def kernel(lengths, W):
    n = lengths.shape[0]
    lengths = lengths.astype(jnp.int32)
    wt = W.astype(jnp.float32).T.reshape(-1)

    mesh = plsc.VectorSubcoreMesh(core_axis_name="c", subcore_axis_name="s")
    out_t = pl.kernel(
        _body,
        out_type=jax.ShapeDtypeStruct((_D, n), jnp.float32),
        mesh=mesh,
        compiler_params=pltpu.CompilerParams(needs_layout_passes=False),
        scratch_types=[
            pltpu.VMEM((_D * 8,), jnp.float32),
            pltpu.VMEM((2, _C), jnp.int32),
            pltpu.VMEM((2, _D, _C), jnp.float32),
            pltpu.SemaphoreType.DMA,
            pltpu.SemaphoreType.DMA,
            pltpu.SemaphoreType.DMA,
            pltpu.SemaphoreType.DMA,
        ],
    )(lengths, wt)
    return out_t.T
